# baseline (speedup 1.0000x reference)
import jax
import jax.numpy as jnp
from jax import lax
from jax.experimental import pallas as pl
from jax.experimental.pallas import tpu as pltpu
from jax.experimental.pallas import tpu_sc as plsc

NUM_USERS = 1000000
NUM_ITEMS = 1000000
EMBED_DIM = 32
BATCH = 16384

NC, NS, L = 2, 16, 16
NW = NC * NS
B_PER_W = BATCH // NW
C = L
PASSES = B_PER_W // C


def _body(user_ref, item_ref, ut_ref, it_ref, out_ref,
          idx_u, idx_i, buf_u, buf_i, out_v, sem_a, sem_b):
  wid = lax.axis_index("s") * NC + lax.axis_index("c")
  base = wid * B_PER_W

  pltpu.sync_copy(user_ref.at[pl.ds(base, B_PER_W)], idx_u)
  pltpu.sync_copy(item_ref.at[pl.ds(base, B_PER_W)], idx_i)

  lane = lax.iota(jnp.int32, L)
  sems = (sem_a, sem_b)

  def fire(p, slot, sem):
    vu = idx_u[pl.ds(p * C, C)]
    vi = idx_i[pl.ds(p * C, C)]
    for t in range(C):
      ru = (vu[t] // L) * 0
      ri = (vi[t] // L) * 0
      pltpu.async_copy(ut_ref.at[:, :, pl.ds(ru, L)],
                       buf_u.at[slot, :, :, pl.ds(t * L, L)], sem)
      pltpu.async_copy(it_ref.at[:, :, pl.ds(ri, L)],
                       buf_i.at[slot, :, :, pl.ds(t * L, L)], sem)

  def drain(slot, sem):
    pltpu.make_async_copy(ut_ref.at[:, :, pl.ds(0, C * L)],
                          buf_u.at[slot], sem).wait()
    pltpu.make_async_copy(it_ref.at[:, :, pl.ds(0, C * L)],
                          buf_i.at[slot], sem).wait()

  def compute(p, slot):
    vu = idx_u[pl.ds(p * C, C)]
    vi = idx_i[pl.ds(p * C, C)]
    lidx_u = lane * L + (vu % L)
    lidx_i = lane * L + (vi % L)
    bu = buf_u.at[slot]
    bi = buf_i.at[slot]
    acc = lidx_u.astype(jnp.float32) + lidx_i.astype(jnp.float32)
    out_v[pl.ds(p * C, C)] = acc

  fire(0, 0, sems[0])

  def two_pass(p2, _):
    p = 2 * p2
    fire(p + 1, 1, sems[1])
    drain(0, sems[0])
    compute(p, 0)

    @pl.when(p2 < PASSES // 2 - 1)
    def _fire_next():
      fire(p + 2, 0, sems[0])

    drain(1, sems[1])
    compute(p + 1, 1)
    return _

  lax.fori_loop(0, PASSES // 2, two_pass, None)

  pltpu.sync_copy(out_v, out_ref.at[pl.ds(base, B_PER_W)])


@jax.jit
def _scores(user_r, item_r, ut3, it3):
  mesh = plsc.VectorSubcoreMesh(core_axis_name="c", subcore_axis_name="s",
                                num_cores=NC, num_subcores=NS)
  return pl.kernel(
      _body,
      out_type=jax.ShapeDtypeStruct((BATCH,), jnp.float32),
      mesh=mesh,
      compiler_params=pltpu.CompilerParams(needs_layout_passes=False,
                                           use_tc_tiling_on_sc=True),
      scratch_types=[
          pltpu.VMEM((B_PER_W,), jnp.int32),
          pltpu.VMEM((B_PER_W,), jnp.int32),
          pltpu.VMEM((2, 4, 8, C * L), jnp.float32),
          pltpu.VMEM((2, 4, 8, C * L), jnp.float32),
          pltpu.VMEM((B_PER_W,), jnp.float32),
          pltpu.SemaphoreType.DMA,
          pltpu.SemaphoreType.DMA,
      ],
  )(user_r, item_r, ut3, it3)


def kernel(user, item, user_table, item_table):
  ut3 = user_table.T.reshape(4, 8, NUM_USERS)
  it3 = item_table.T.reshape(4, 8, NUM_ITEMS)
  return _scores(user.astype(jnp.int32), item.astype(jnp.int32), ut3, it3)


# speedup vs baseline: 1.5852x; 1.5852x over previous
import jax
import jax.numpy as jnp
from jax import lax
from jax.experimental import pallas as pl
from jax.experimental.pallas import tpu as pltpu
from jax.experimental.pallas import tpu_sc as plsc

NUM_USERS = 1000000
NUM_ITEMS = 1000000
EMBED_DIM = 32
BATCH = 16384

NC, NS, L = 2, 16, 16
NW = NC * NS
B_PER_W = BATCH // NW
C = 64
PASSES = B_PER_W // C


def _body(user_ref, item_ref, ut_ref, it_ref, out_ref,
          idx_u, idx_i, buf_u, buf_i, out_v, sem):
  wid = lax.axis_index("s") * NC + lax.axis_index("c")
  base = wid * B_PER_W

  pltpu.sync_copy(user_ref.at[pl.ds(base, B_PER_W)], idx_u)
  pltpu.sync_copy(item_ref.at[pl.ds(base, B_PER_W)], idx_i)

  lane = lax.iota(jnp.int32, L)

  def one_pass(p, _):
    pb = p * C

    def fire_group(gg, _):
      vu = idx_u[pl.ds(pb + gg * L, L)]
      vi = idx_i[pl.ds(pb + gg * L, L)]
      for t in range(L):
        dst = (gg * L + t) * L
        ru = (vu[t] // L) * L
        ri = (vi[t] // L) * L
        pltpu.async_copy(ut_ref.at[:, :, pl.ds(ru, L)],
                         buf_u.at[:, :, pl.ds(dst, L)], sem)
        pltpu.async_copy(it_ref.at[:, :, pl.ds(ri, L)],
                         buf_i.at[:, :, pl.ds(dst, L)], sem)
      return _

    lax.fori_loop(0, C // L, fire_group, None)

    pltpu.make_async_copy(ut_ref.at[:, :, pl.ds(0, C * L)],
                          buf_u, sem).wait()
    pltpu.make_async_copy(it_ref.at[:, :, pl.ds(0, C * L)],
                          buf_i, sem).wait()

    def block(blk, _):
      b0 = pb + blk * L
      vu = idx_u[pl.ds(b0, L)]
      vi = idx_i[pl.ds(b0, L)]
      lidx_u = (blk * L + lane) * L + (vu % L)
      lidx_i = (blk * L + lane) * L + (vi % L)
      acc = jnp.zeros((L,), jnp.float32)
      for i in range(4):
        ii = jnp.full((L,), i, jnp.int32)
        for s in range(8):
          ss = jnp.full((L,), s, jnp.int32)
          u = plsc.load_gather(buf_u, [ii, ss, lidx_u])
          v = plsc.load_gather(buf_i, [ii, ss, lidx_i])
          acc = acc + u * v
      out_v[pl.ds(b0, L)] = acc
      return _

    lax.fori_loop(0, C // L, block, None)
    return _

  lax.fori_loop(0, PASSES, one_pass, None)

  pltpu.sync_copy(out_v, out_ref.at[pl.ds(base, B_PER_W)])


@jax.jit
def _scores(user_r, item_r, ut3, it3):
  mesh = plsc.VectorSubcoreMesh(core_axis_name="c", subcore_axis_name="s",
                                num_cores=NC, num_subcores=NS)
  return pl.kernel(
      _body,
      out_type=jax.ShapeDtypeStruct((BATCH,), jnp.float32),
      mesh=mesh,
      compiler_params=pltpu.CompilerParams(needs_layout_passes=False,
                                           use_tc_tiling_on_sc=True),
      scratch_types=[
          pltpu.VMEM((B_PER_W,), jnp.int32),
          pltpu.VMEM((B_PER_W,), jnp.int32),
          pltpu.VMEM((4, 8, C * L), jnp.float32),
          pltpu.VMEM((4, 8, C * L), jnp.float32),
          pltpu.VMEM((B_PER_W,), jnp.float32),
          pltpu.SemaphoreType.DMA,
      ],
  )(user_r, item_r, ut3, it3)


def kernel(user, item, user_table, item_table):
  ut3 = user_table.T.reshape(4, 8, NUM_USERS)
  it3 = item_table.T.reshape(4, 8, NUM_ITEMS)
  return _scores(user.astype(jnp.int32), item.astype(jnp.int32), ut3, it3)
